# baseline (speedup 1.0000x reference)
import functools

import jax
import jax.numpy as jnp
from jax.experimental import pallas as pl
from jax.experimental.pallas import tpu as pltpu

H = 1024
F = 4096
E = 8
T = 2048
TF = 512
NF = F // TF


def _moe_body(x_ref, wr_ref, wg_ref, wu_ref, wd_ref, out_ref, w_out_ref,
              xb_ref):
    e = pl.program_id(0)
    f = pl.program_id(1)

    @pl.when((e == 0) & (f == 0))
    def _init():
        w = jax.nn.relu(
            jax.lax.dot_general(x_ref[...], wr_ref[...],
                                (((1,), (1,)), ((), ())),
                                preferred_element_type=jnp.float32))
        w_out_ref[...] = w
        out_ref[...] = jnp.zeros_like(out_ref)
        xb_ref[...] = x_ref[...].astype(jnp.bfloat16)

    x = xb_ref[...]
    g = jax.lax.dot_general(x, wg_ref[0], (((1,), (1,)), ((), ())),
                            preferred_element_type=jnp.float32)
    u = jax.lax.dot_general(x, wu_ref[0], (((1,), (1,)), ((), ())),
                            preferred_element_type=jnp.float32)
    a = (g * jax.nn.sigmoid(g) * u).astype(jnp.bfloat16)
    part = jax.lax.dot_general(a, wd_ref[0], (((1,), (1,)), ((), ())),
                               preferred_element_type=jnp.float32)
    sel = (jax.lax.broadcasted_iota(jnp.int32, (1, E), 1) == e)
    w_col = jnp.sum(jnp.where(sel, w_out_ref[...], 0.0), axis=1, keepdims=True)
    out_ref[...] += part * w_col


def _moe(x, Wr, Wg, Wu, Wd):
    out, w = pl.pallas_call(
        _moe_body,
        grid=(E, NF),
        in_specs=[
            pl.BlockSpec((T, H), lambda e, f: (0, 0)),
            pl.BlockSpec((E, H), lambda e, f: (0, 0)),
            pl.BlockSpec((1, TF, H), lambda e, f: (e, f, 0)),
            pl.BlockSpec((1, TF, H), lambda e, f: (e, f, 0)),
            pl.BlockSpec((1, H, TF), lambda e, f: (e, 0, f)),
        ],
        out_specs=[
            pl.BlockSpec((T, H), lambda e, f: (0, 0)),
            pl.BlockSpec((T, E), lambda e, f: (0, 0)),
        ],
        out_shape=[
            jax.ShapeDtypeStruct((T, H), jnp.float32),
            jax.ShapeDtypeStruct((T, E), jnp.float32),
        ],
        scratch_shapes=[pltpu.VMEM((T, H), jnp.bfloat16)],
        compiler_params=pltpu.CompilerParams(
            dimension_semantics=("arbitrary", "arbitrary"),
            vmem_limit_bytes=120 * 1024 * 1024,
        ),
    )(x, Wr, Wg, Wu, Wd)
    return out, w


def kernel(hidden_states, Wr, Wg, Wu, Wd):
    orig_shape = hidden_states.shape
    x = hidden_states.reshape(-1, orig_shape[-1])
    out, w = _moe(x, Wr,
                  Wg.astype(jnp.bfloat16),
                  Wu.astype(jnp.bfloat16),
                  Wd.astype(jnp.bfloat16))
    return (out.reshape(orig_shape), w.reshape(orig_shape[:-1] + (E,)))


# speedup vs baseline: 1.1071x; 1.1071x over previous
import functools

import jax
import jax.numpy as jnp
from jax import lax
from jax.experimental import pallas as pl
from jax.experimental.pallas import tpu as pltpu
from jax.experimental.pallas import tpu_sc as plsc

H = 1024
F = 4096
E = 8
T = 2048

TF = 512
NF = F // TF
TM = 256
NT = T // TM
CH = 64
NW = 32
WPE = NW // E
CPW = (T // CH) // WPE
TPW = T // NW
SUB = 16
NSUB = TPW // SUB


def _router_body(x_ref, wr_ref, w_ref, wt_ref):
    logits = jax.lax.dot_general(x_ref[...], wr_ref[...],
                                 (((1,), (1,)), ((), ())),
                                 preferred_element_type=jnp.float32)
    w_ref[...] = jax.nn.relu(logits)
    logits_t = jax.lax.dot_general(wr_ref[...], x_ref[...],
                                   (((1,), (1,)), ((), ())),
                                   preferred_element_type=jnp.float32)
    wt_ref[...] = jax.nn.relu(logits_t)


def _router(x, Wr):
    return pl.pallas_call(
        _router_body,
        out_shape=[
            jax.ShapeDtypeStruct((T, E), jnp.float32),
            jax.ShapeDtypeStruct((E, T), jnp.float32),
        ],
    )(x, Wr)


_mesh = plsc.VectorSubcoreMesh(core_axis_name="c", subcore_axis_name="s")


@functools.partial(
    pl.kernel, mesh=_mesh,
    out_type=[
        jax.ShapeDtypeStruct((E, 16), jnp.int32),
        jax.ShapeDtypeStruct((E, T), jnp.int32),
        jax.ShapeDtypeStruct((E * T, H), jnp.float32),
    ],
    scratch_types=[
        pltpu.VMEM((T,), jnp.float32),
        pltpu.VMEM((T,), jnp.int32),
        pltpu.VMEM((T,), jnp.int32),
        pltpu.VMEM((16,), jnp.int32),
        pltpu.VMEM((CH, H), jnp.float32),
        pltpu.SemaphoreType.DMA,
    ],
    compiler_params=pltpu.CompilerParams(needs_layout_passes=False),
)
def _dispatch(wt_hbm, x_hbm, cnt_hbm, pos_hbm, xg_hbm,
              wvec, idxbuf, posbuf, cntbuf, rows, sem):
    c = lax.axis_index("c")
    s = lax.axis_index("s")
    wid = s * 2 + c
    e = wid % E
    q = wid // E

    pltpu.sync_copy(wt_hbm.at[e], wvec)

    def _zero(i, carry):
        idxbuf[pl.ds(i * 16, 16)] = jnp.zeros((16,), jnp.int32)
        return carry

    lax.fori_loop(0, T // 16, _zero, jnp.int32(0))

    ones16 = jnp.ones((16,), jnp.int32)
    zeros16 = jnp.zeros((16,), jnp.int32)

    def _scan(i, carry):
        v = wvec[pl.ds(i * 16, 16)]
        m = v > 0.0
        inc = plsc.cumsum(jnp.where(m, ones16, zeros16))
        pos_v = inc + carry - ones16
        tok = lax.iota(jnp.int32, 16) + jnp.full((16,), i * 16, jnp.int32)
        plsc.store_scatter(idxbuf, [pos_v], tok, mask=m)
        posbuf[pl.ds(i * 16, 16)] = jnp.where(m, pos_v, zeros16)
        return carry + plsc.all_reduce_population_count(m)

    cnt_vec = lax.fori_loop(0, T // 16, _scan, jnp.zeros((16,), jnp.int32))
    cnt = cnt_vec[0]

    @pl.when(q == 0)
    def _write_meta():
        cntbuf[...] = cnt_vec
        pltpu.sync_copy(cntbuf, cnt_hbm.at[e])
        pltpu.sync_copy(posbuf, pos_hbm.at[e])

    for k in range(CPW):
        base = (q + k * WPE) * CH

        @pl.when(base < cnt)
        def _gather(base=base):
            pltpu.async_copy(x_hbm.at[idxbuf.at[pl.ds(base, CH)]],
                             rows, sem).wait()
            pltpu.sync_copy(rows, xg_hbm.at[pl.ds(e * T + base, CH)])


def _experts_body(cnt_ref, xg_ref, wg_ref, wu_ref, wd_ref, og_ref):
    e = pl.program_id(0)
    f = pl.program_id(1)
    count = cnt_ref[e, 0]
    for s in range(NT):

        @pl.when(s * TM < count)
        def _tile(s=s):
            x_s = xg_ref[s * TM:(s + 1) * TM, :]
            g = jax.lax.dot_general(x_s, wg_ref[0], (((1,), (1,)), ((), ())),
                                    preferred_element_type=jnp.float32)
            u = jax.lax.dot_general(x_s, wu_ref[0], (((1,), (1,)), ((), ())),
                                    preferred_element_type=jnp.float32)
            a = g * jax.nn.sigmoid(g) * u
            part = jax.lax.dot_general(a, wd_ref[0], (((1,), (1,)), ((), ())),
                                       preferred_element_type=jnp.float32)

            @pl.when(f == 0)
            def _set():
                og_ref[s * TM:(s + 1) * TM, :] = part

            @pl.when(f != 0)
            def _add():
                og_ref[s * TM:(s + 1) * TM, :] += part


def _experts(cnt, xg, Wg, Wu, Wd):
    return pl.pallas_call(
        _experts_body,
        grid=(E, NF),
        in_specs=[
            pl.BlockSpec(memory_space=pltpu.SMEM),
            pl.BlockSpec((T, H), lambda e, f: (e, 0)),
            pl.BlockSpec((1, TF, H), lambda e, f: (e, f, 0)),
            pl.BlockSpec((1, TF, H), lambda e, f: (e, f, 0)),
            pl.BlockSpec((1, H, TF), lambda e, f: (e, 0, f)),
        ],
        out_specs=pl.BlockSpec((T, H), lambda e, f: (e, 0)),
        out_shape=jax.ShapeDtypeStruct((E * T, H), jnp.float32),
        compiler_params=pltpu.CompilerParams(
            dimension_semantics=("arbitrary", "arbitrary"),
            vmem_limit_bytes=120 * 1024 * 1024,
        ),
    )(cnt, xg, Wg, Wu, Wd)


@functools.partial(
    pl.kernel, mesh=_mesh,
    out_type=jax.ShapeDtypeStruct((T, H), jnp.float32),
    scratch_types=[
        pltpu.VMEM((E, TPW), jnp.int32),
        pltpu.VMEM((E, TPW), jnp.float32),
        pltpu.VMEM((E * SUB,), jnp.int32),
        pltpu.VMEM((CH, H), jnp.float32),
        pltpu.VMEM((SUB, H), jnp.float32),
        pltpu.SemaphoreType.DMA,
    ],
    compiler_params=pltpu.CompilerParams(needs_layout_passes=False),
)
def _combine(og_hbm, pos_hbm, wt_hbm, out_hbm, posv, wv, gix, rows, acc, sem):
    c = lax.axis_index("c")
    s = lax.axis_index("s")
    wid = s * 2 + c
    tbase = wid * TPW

    for ee in range(E):
        pltpu.sync_copy(pos_hbm.at[ee].at[pl.ds(tbase, TPW)], posv.at[ee])
        pltpu.sync_copy(wt_hbm.at[ee].at[pl.ds(tbase, TPW)], wv.at[ee])

    fzeros16 = jnp.zeros((16,), jnp.float32)
    for sub in range(NSUB):
        for ee in range(E):
            pvec = posv[ee, pl.ds(sub * SUB, SUB)]
            gix[pl.ds(ee * SUB, SUB)] = pvec + jnp.full((16,), ee * T,
                                                        jnp.int32)

        for hh in range(2):
            pltpu.async_copy(og_hbm.at[gix.at[pl.ds(hh * CH, CH)]],
                             rows, sem).wait()
            wrows = [wv[4 * hh + el, pl.ds(sub * SUB, SUB)] for el in range(4)]

            for i in range(SUB):
                wspl = [jnp.full((16,), wrows[el][i], jnp.float32)
                        for el in range(4)]

                def _chunk(k, carry, i=i, hh=hh, wspl=wspl):
                    sl = pl.ds(k * 16, 16)
                    if hh == 0:
                        accv = fzeros16
                    else:
                        accv = acc[i, sl]
                    for el in range(4):
                        rowv = rows[el * SUB + i, sl]
                        accv = accv + jnp.where(wspl[el] > 0.0,
                                                rowv * wspl[el], fzeros16)
                    acc[i, sl] = accv
                    return carry

                lax.fori_loop(0, H // 16, _chunk, jnp.int32(0))

        pltpu.sync_copy(acc, out_hbm.at[pl.ds(tbase + sub * SUB, SUB)])


def kernel(hidden_states, Wr, Wg, Wu, Wd):
    orig_shape = hidden_states.shape
    x = hidden_states.reshape(T, H)
    w, wt = _router(x, Wr)
    cnt, pos, xg = _dispatch(wt, x)
    og = _experts(cnt, xg, Wg, Wu, Wd)
    out = _combine(og, pos, wt)
    return (out.reshape(orig_shape), w.reshape(orig_shape[:-1] + (E,)))
